# baseline (speedup 1.0000x reference)
import functools

import jax
import jax.numpy as jnp
from jax import lax
from jax.experimental import pallas as pl
from jax.experimental.pallas import tpu as pltpu
from jax.experimental.pallas import tpu_sc as plsc

N, E, H, De, L = 10000, 320000, 128, 16, 3
C = 64
NT = 16
NT2 = 2 * NT
NCH = 158
NCH0, NCH1 = 224, 92
E2 = NT2 * C * NCH
ETT = E2 // NT2
NROWS = N + 16
XB = 624
XREM = N - NT * XB
AREM = NROWS - NT * XB
BE = 2048
NEB = E2 // BE


NEBR = E // BE
ETAIL = E - NEBR * BE


def _ea_body(a_ref, t_ref, w_ref, b_ref, o_ref):
    e = pl.program_id(0)
    a = jnp.where(e >= NEBR, t_ref[...], a_ref[...])
    o_ref[...] = jnp.dot(a, w_ref[...],
                         preferred_element_type=jnp.float32) + b_ref[...]


def _compute_ea(edge_attr, tail, w_l, b_l):
    return pl.pallas_call(
        _ea_body,
        grid=(NEB,),
        in_specs=[
            pl.BlockSpec((BE, De), lambda e: (jnp.minimum(e, NEBR - 1), 0)),
            pl.BlockSpec((BE, De), lambda e: (jnp.clip(e - NEBR, 0, 1), 0)),
            pl.BlockSpec((De, H), lambda e: (0, 0)),
            pl.BlockSpec((1, H), lambda e: (0, 0)),
        ],
        out_specs=pl.BlockSpec((BE, H), lambda e: (e, 0)),
        out_shape=jax.ShapeDtypeStruct((E2, H), jnp.float32),
    )(edge_attr, tail, w_l, b_l)


def _make_sc_layer():
    mesh = plsc.VectorSubcoreMesh(core_axis_name="c", subcore_axis_name="s")

    @functools.partial(
        pl.kernel,
        mesh=mesh,
        out_type=jax.ShapeDtypeStruct((2, N, H), jnp.float32),
        scratch_types=[
            pltpu.VMEM_SHARED((NROWS, H), jnp.float32),
            pltpu.VMEM((2, C), jnp.int32),
            pltpu.VMEM((2, C), jnp.int32),
            pltpu.VMEM((2, C, H), jnp.float32),
            pltpu.VMEM((2, C, H), jnp.float32),
            pltpu.SemaphoreType.DMA,
            pltpu.SemaphoreType.DMA,
            pltpu.SemaphoreType.DMA,
            pltpu.SemaphoreType.DMA,
            pltpu.SemaphoreType.DMA,
            pltpu.SemaphoreType.DMA,
        ],
    )
    def sc_layer(x_hbm, ea_hbm, src_hbm, dst_hbm, zero_hbm, out_hbm,
                 aggr_s, srcb, dstb, eab, rowb,
                 sem_i0, sem_i1, sem_e0, sem_e1, sem_g0, sem_g1):
        c = lax.axis_index("c")
        s = lax.axis_index("s")
        sems_i = (sem_i0, sem_i1)
        sems_e = (sem_e0, sem_e1)
        sems_g = (sem_g0, sem_g1)
        ebase = jnp.where(c == 0, s * (NCH0 * C),
                          NT * (NCH0 * C) + s * (NCH1 * C))
        nch = jnp.where(c == 0, NCH0, NCH1)

        pltpu.sync_copy(zero_hbm.at[pl.ds(s * XB, XB)], aggr_s.at[pl.ds(s * XB, XB)])

        @pl.when(s == NT - 1)
        def _():
            pltpu.sync_copy(zero_hbm.at[pl.ds(NT * XB, AREM)],
                            aggr_s.at[pl.ds(NT * XB, AREM)])

        plsc.subcore_barrier()

        def start_src(k, b):
            pltpu.async_copy(src_hbm.at[pl.ds(ebase + k * C, C)], srcb.at[b],
                             sems_i[b])

        def start_dst(k, b):
            pltpu.async_copy(dst_hbm.at[pl.ds(ebase + k * C, C)], dstb.at[b],
                             sems_i[b])

        def start_dat(k, b):
            pltpu.make_async_copy(src_hbm.at[pl.ds(ebase + k * C, C)],
                                  srcb.at[b], sems_i[b]).wait()
            pltpu.make_async_copy(dst_hbm.at[pl.ds(ebase + k * C, C)],
                                  dstb.at[b], sems_i[b]).wait()
            pltpu.async_copy(x_hbm.at[srcb.at[b]], rowb.at[b], sems_g[b])
            pltpu.async_copy(ea_hbm.at[pl.ds(ebase + k * C, C)],
                             eab.at[b], sems_e[b])

        def finish(k, b):
            pltpu.make_async_copy(x_hbm.at[srcb.at[b]], rowb.at[b],
                                  sems_g[b]).wait()
            pltpu.make_async_copy(ea_hbm.at[pl.ds(ebase + k * C, C)],
                                  eab.at[b], sems_e[b]).wait()

        def comp(k, b):
            rb = rowb.at[b]
            eb = eab.at[b]

            @plsc.parallel_loop(0, C // 8, unroll=2)
            def _(i8):
                for g in range(8):
                    for j in range(8):
                        sl = pl.ds(j * 16, 16)
                        v = rb[i8 * 8 + g, sl] + eb[i8 * 8 + g, sl]
                        rb[i8 * 8 + g, sl] = jnp.maximum(v, 0.0)

            pltpu.sync_copy(rb, aggr_s.at[dstb.at[b]], add=True)

        start_src(0, 0)
        start_dst(0, 0)
        start_src(1, 1)
        start_dst(1, 1)
        start_dat(0, 0)

        @pl.loop(0, nch - 1, step=2)
        def _(k):
            start_dat(k + 1, 1)
            finish(k, 0)

            @pl.when(k + 2 < nch)
            def _():
                start_src(k + 2, 0)

            comp(k, 0)

            @pl.when(k + 2 < nch)
            def _():
                start_dst(k + 2, 0)

            finish(k + 1, 1)

            @pl.when(k + 3 < nch)
            def _():
                start_src(k + 3, 1)

            comp(k + 1, 1)

            @pl.when(k + 3 < nch)
            def _():
                start_dst(k + 3, 1)

            @pl.when(k + 2 < nch)
            def _():
                start_dat(k + 2, 0)

        plsc.subcore_barrier()
        pltpu.sync_copy(aggr_s.at[pl.ds(s * XB, XB)],
                        out_hbm.at[c, pl.ds(s * XB, XB)])

        @pl.when(s == NT - 1)
        def _():
            pltpu.sync_copy(aggr_s.at[pl.ds(NT * XB, XREM)],
                            out_hbm.at[c, pl.ds(NT * XB, XREM)])

    return sc_layer


def _dense_body(x_ref, ag_ref, w1_ref, b1_ref, w2_ref, b2_ref, g_ref, bt_ref,
                xo_ref):
    x = x_ref[...]
    out = x + ag_ref[0] + ag_ref[1]
    h = jnp.dot(out, w1_ref[...], preferred_element_type=jnp.float32) + b1_ref[0]
    h = jnp.maximum(h, 0.0)
    h = jnp.dot(h, w2_ref[...], preferred_element_type=jnp.float32) + b2_ref[0]
    h = jnp.maximum(h, 0.0)
    mean = jnp.mean(h, axis=0, keepdims=True)
    var = jnp.mean((h - mean) ** 2, axis=0, keepdims=True)
    h = (h - mean) / jnp.sqrt(var + 1e-5) * g_ref[0] + bt_ref[0]
    xo_ref[...] = x + h


def _dense_layer(x, aggr, w1, b1, w2, b2, g, bt):
    return pl.pallas_call(
        _dense_body,
        out_shape=jax.ShapeDtypeStruct((N, H), jnp.float32),
    )(x, aggr, w1, b1.reshape(1, H), w2, b2.reshape(1, H),
      g.reshape(1, H), bt.reshape(1, H))


def kernel(x, edge_index, edge_attr, We, be, W1, b1, W2, b2, gamma, beta):
    pad = E2 - E
    src = jnp.concatenate([edge_index[0], jnp.zeros((pad,), jnp.int32)])
    dst = jnp.concatenate([edge_index[1], jnp.full((pad,), N, jnp.int32)])

    zeros_rows = jnp.zeros((NROWS, H), jnp.float32)
    sc_layer = _make_sc_layer()
    tail = jnp.pad(edge_attr[NEBR * BE:], ((0, 2 * BE - ETAIL), (0, 0)))

    for l in range(L):
        ea_l = _compute_ea(edge_attr, tail, We[l], be[l].reshape(1, H))
        aggr = sc_layer(x, ea_l, src, dst, zeros_rows)
        x = _dense_layer(x, aggr, W1[l], b1[l], W2[l], b2[l],
                         gamma[l], beta[l])
    return x


# speedup vs baseline: 1.0531x; 1.0531x over previous
import functools

import jax
import jax.numpy as jnp
from jax import lax
from jax.experimental import pallas as pl
from jax.experimental.pallas import tpu as pltpu
from jax.experimental.pallas import tpu_sc as plsc

N, E, H, De, L = 10000, 320000, 128, 16, 3
C = 64
NT = 16
NT2 = 2 * NT
NCH = 158
NCH0, NCH1 = 208, 108
E2 = NT2 * C * NCH
ETT = E2 // NT2
NROWS = N + 16
XB = 624
XREM = N - NT * XB
AREM = NROWS - NT * XB
BE = 2048
NEB = E2 // BE


NEBR = E // BE
ETAIL = E - NEBR * BE


def _ea_body(a_ref, t_ref, w_ref, b_ref, o_ref):
    e = pl.program_id(0)
    a = jnp.where(e >= NEBR, t_ref[...], a_ref[...])
    o_ref[...] = jnp.dot(a, w_ref[...],
                         preferred_element_type=jnp.float32) + b_ref[...]


def _compute_ea(edge_attr, tail, w_l, b_l):
    return pl.pallas_call(
        _ea_body,
        grid=(NEB,),
        in_specs=[
            pl.BlockSpec((BE, De), lambda e: (jnp.minimum(e, NEBR - 1), 0)),
            pl.BlockSpec((BE, De), lambda e: (jnp.clip(e - NEBR, 0, 1), 0)),
            pl.BlockSpec((De, H), lambda e: (0, 0)),
            pl.BlockSpec((1, H), lambda e: (0, 0)),
        ],
        out_specs=pl.BlockSpec((BE, H), lambda e: (e, 0)),
        out_shape=jax.ShapeDtypeStruct((E2, H), jnp.float32),
    )(edge_attr, tail, w_l, b_l)


def _make_sc_layer():
    mesh = plsc.VectorSubcoreMesh(core_axis_name="c", subcore_axis_name="s")

    @functools.partial(
        pl.kernel,
        mesh=mesh,
        out_type=jax.ShapeDtypeStruct((2, N, H), jnp.float32),
        scratch_types=[
            pltpu.VMEM_SHARED((NROWS, H), jnp.float32),
            pltpu.VMEM((2, C), jnp.int32),
            pltpu.VMEM((2, C), jnp.int32),
            pltpu.VMEM((2, C, H), jnp.float32),
            pltpu.VMEM((2, C, H), jnp.float32),
            pltpu.SemaphoreType.DMA,
            pltpu.SemaphoreType.DMA,
            pltpu.SemaphoreType.DMA,
            pltpu.SemaphoreType.DMA,
            pltpu.SemaphoreType.DMA,
            pltpu.SemaphoreType.DMA,
        ],
    )
    def sc_layer(x_hbm, ea_hbm, src_hbm, dst_hbm, zero_hbm, out_hbm,
                 aggr_s, srcb, dstb, eab, rowb,
                 sem_i0, sem_i1, sem_e0, sem_e1, sem_g0, sem_g1):
        c = lax.axis_index("c")
        s = lax.axis_index("s")
        sems_i = (sem_i0, sem_i1)
        sems_e = (sem_e0, sem_e1)
        sems_g = (sem_g0, sem_g1)
        ebase = jnp.where(c == 0, s * (NCH0 * C),
                          NT * (NCH0 * C) + s * (NCH1 * C))
        nch = jnp.where(c == 0, NCH0, NCH1)

        pltpu.sync_copy(zero_hbm.at[pl.ds(s * XB, XB)], aggr_s.at[pl.ds(s * XB, XB)])

        @pl.when(s == NT - 1)
        def _():
            pltpu.sync_copy(zero_hbm.at[pl.ds(NT * XB, AREM)],
                            aggr_s.at[pl.ds(NT * XB, AREM)])

        plsc.subcore_barrier()

        def start_src(k, b):
            pltpu.async_copy(src_hbm.at[pl.ds(ebase + k * C, C)], srcb.at[b],
                             sems_i[b])

        def start_dst(k, b):
            pltpu.async_copy(dst_hbm.at[pl.ds(ebase + k * C, C)], dstb.at[b],
                             sems_i[b])

        def start_dat(k, b):
            pltpu.make_async_copy(src_hbm.at[pl.ds(ebase + k * C, C)],
                                  srcb.at[b], sems_i[b]).wait()
            pltpu.make_async_copy(dst_hbm.at[pl.ds(ebase + k * C, C)],
                                  dstb.at[b], sems_i[b]).wait()
            pltpu.async_copy(x_hbm.at[srcb.at[b]], rowb.at[b], sems_g[b])
            pltpu.async_copy(ea_hbm.at[pl.ds(ebase + k * C, C)],
                             eab.at[b], sems_e[b])

        def finish(k, b):
            pltpu.make_async_copy(x_hbm.at[srcb.at[b]], rowb.at[b],
                                  sems_g[b]).wait()
            pltpu.make_async_copy(ea_hbm.at[pl.ds(ebase + k * C, C)],
                                  eab.at[b], sems_e[b]).wait()

        def comp(k, b):
            rb = rowb.at[b]
            eb = eab.at[b]

            @plsc.parallel_loop(0, C // 8, unroll=2)
            def _(i8):
                for g in range(8):
                    for j in range(8):
                        sl = pl.ds(j * 16, 16)
                        v = rb[i8 * 8 + g, sl] + eb[i8 * 8 + g, sl]
                        rb[i8 * 8 + g, sl] = jnp.maximum(v, 0.0)

            pltpu.sync_copy(rb, aggr_s.at[dstb.at[b]], add=True)

        start_src(0, 0)
        start_dst(0, 0)
        start_src(1, 1)
        start_dst(1, 1)
        start_dat(0, 0)

        @pl.loop(0, nch - 1, step=2)
        def _(k):
            start_dat(k + 1, 1)
            finish(k, 0)

            @pl.when(k + 2 < nch)
            def _():
                start_src(k + 2, 0)

            comp(k, 0)

            @pl.when(k + 2 < nch)
            def _():
                start_dst(k + 2, 0)

            finish(k + 1, 1)

            @pl.when(k + 3 < nch)
            def _():
                start_src(k + 3, 1)

            comp(k + 1, 1)

            @pl.when(k + 3 < nch)
            def _():
                start_dst(k + 3, 1)

            @pl.when(k + 2 < nch)
            def _():
                start_dat(k + 2, 0)

        plsc.subcore_barrier()
        pltpu.sync_copy(aggr_s.at[pl.ds(s * XB, XB)],
                        out_hbm.at[c, pl.ds(s * XB, XB)])

        @pl.when(s == NT - 1)
        def _():
            pltpu.sync_copy(aggr_s.at[pl.ds(NT * XB, XREM)],
                            out_hbm.at[c, pl.ds(NT * XB, XREM)])

    return sc_layer


def _dense_body(x_ref, ag_ref, w1_ref, b1_ref, w2_ref, b2_ref, g_ref, bt_ref,
                xo_ref):
    x = x_ref[...]
    out = x + ag_ref[0] + ag_ref[1]
    h = jnp.dot(out, w1_ref[...], preferred_element_type=jnp.float32) + b1_ref[0]
    h = jnp.maximum(h, 0.0)
    h = jnp.dot(h, w2_ref[...], preferred_element_type=jnp.float32) + b2_ref[0]
    h = jnp.maximum(h, 0.0)
    mean = jnp.mean(h, axis=0, keepdims=True)
    var = jnp.mean((h - mean) ** 2, axis=0, keepdims=True)
    h = (h - mean) / jnp.sqrt(var + 1e-5) * g_ref[0] + bt_ref[0]
    xo_ref[...] = x + h


def _dense_layer(x, aggr, w1, b1, w2, b2, g, bt):
    return pl.pallas_call(
        _dense_body,
        out_shape=jax.ShapeDtypeStruct((N, H), jnp.float32),
    )(x, aggr, w1, b1.reshape(1, H), w2, b2.reshape(1, H),
      g.reshape(1, H), bt.reshape(1, H))


def kernel(x, edge_index, edge_attr, We, be, W1, b1, W2, b2, gamma, beta):
    pad = E2 - E
    src = jnp.concatenate([edge_index[0], jnp.zeros((pad,), jnp.int32)])
    dst = jnp.concatenate([edge_index[1], jnp.full((pad,), N, jnp.int32)])

    zeros_rows = jnp.zeros((NROWS, H), jnp.float32)
    sc_layer = _make_sc_layer()
    tail = jnp.pad(edge_attr[NEBR * BE:], ((0, 2 * BE - ETAIL), (0, 0)))

    for l in range(L):
        ea_l = _compute_ea(edge_attr, tail, We[l], be[l].reshape(1, H))
        aggr = sc_layer(x, ea_l, src, dst, zeros_rows)
        x = _dense_layer(x, aggr, W1[l], b1[l], W2[l], b2[l],
                         gamma[l], beta[l])
    return x
